# Initial kernel scaffold; baseline (speedup 1.0000x reference)
import jax
import jax.numpy as jnp
from jax.experimental import pallas as pl


def kernel(entity_emb, relation_emb, layer_params, edge_conf, edge_index, edge_type):
    raise NotImplementedError("write your pallas kernel here")


# speedup vs baseline: 1.4338x; 1.4338x over previous
import functools

import jax
import jax.numpy as jnp
from jax import lax
from jax.experimental import pallas as pl
from jax.experimental.pallas import tpu as pltpu
from jax.experimental.pallas import tpu_sc as plsc

N_NODES = 10000
N_EDGES = 160000
D = 256
H = 128
N_REL = 64

NC = 2
NS = 16
NW = NC * NS
L = 16

C = 32
NCHUNK = 160
Q = 64
EPW = C * NCHUNK
EP = EPW * NW
NPAD = 10240
MBLK = 1024


def _mm_body(x_ref, w_ref, b_ref, u_ref, wd_ref, xv0_ref, xv1_ref, w0x_ref):
    p = jnp.dot(x_ref[...], w_ref[...], preferred_element_type=jnp.float32)
    p = p + b_ref[...]
    u_ref[...] = p[:, 0:256]
    wd_ref[...] = p[:, 256:512]
    xv0_ref[...] = p[:, 512:640]
    xv1_ref[...] = p[:, 640:768]
    w0x_ref[...] = p[:, 768:1024]


def _node_matmul(x, wcat, bcat):
    n = NPAD // MBLK
    outs = (
        jax.ShapeDtypeStruct((NPAD, 256), jnp.float32),
        jax.ShapeDtypeStruct((NPAD, 256), jnp.float32),
        jax.ShapeDtypeStruct((NPAD, H), jnp.float32),
        jax.ShapeDtypeStruct((NPAD, H), jnp.float32),
        jax.ShapeDtypeStruct((NPAD, 256), jnp.float32),
    )
    return pl.pallas_call(
        _mm_body,
        grid=(n,),
        in_specs=[
            pl.BlockSpec((MBLK, 256), lambda i: (i, 0)),
            pl.BlockSpec((256, 1024), lambda i: (0, 0)),
            pl.BlockSpec((1, 1024), lambda i: (0, 0)),
        ],
        out_specs=[
            pl.BlockSpec((MBLK, 256), lambda i: (i, 0)),
            pl.BlockSpec((MBLK, 256), lambda i: (i, 0)),
            pl.BlockSpec((MBLK, H), lambda i: (i, 0)),
            pl.BlockSpec((MBLK, H), lambda i: (i, 0)),
            pl.BlockSpec((MBLK, 256), lambda i: (i, 0)),
        ],
        out_shape=outs,
    )(x, wcat, bcat)


def _vt_body(r_ref, a_ref, b_ref, o_ref):
    o_ref[...] = (
        jnp.dot(r_ref[...], a_ref[...], preferred_element_type=jnp.float32)
        + b_ref[...]
    )


def _rel_proj(rel_emb, a_r, b1):
    return pl.pallas_call(
        _vt_body,
        out_shape=jax.ShapeDtypeStruct((N_REL, 256), jnp.float32),
    )(rel_emb, a_r, b1)


def _gate_body(c_ref, a_ref, b_ref, o_ref):
    t = c_ref[...] * a_ref[0, 0] + b_ref[0, 0]
    o_ref[...] = 1.0 / (1.0 + jnp.exp(-t))


def _edge_gate(conf_p, cg_w, cg_b):
    return pl.pallas_call(
        _gate_body,
        out_shape=jax.ShapeDtypeStruct(conf_p.shape, jnp.float32),
    )(conf_p, cg_w, cg_b)


def _fin_body(agg_ref, d_ref, w0x_ref, o_ref):
    den = d_ref[...] + 1e-16
    s = agg_ref[0, 0] + agg_ref[1, 0]
    o_ref[...] = jnp.maximum(s / den + w0x_ref[...], 0.0)


def _finalize(aggp, d, w0x):
    n = NPAD // MBLK
    return pl.pallas_call(
        _fin_body,
        grid=(n, 2),
        in_specs=[
            pl.BlockSpec((2, 1, MBLK, H), lambda i, h: (0, h, i, 0)),
            pl.BlockSpec((MBLK, 1), lambda i, h: (i, 0)),
            pl.BlockSpec((MBLK, H), lambda i, h: (i, h)),
        ],
        out_specs=pl.BlockSpec((MBLK, H), lambda i, h: (i, h)),
        out_shape=jax.ShapeDtypeStruct((NPAD, 256), jnp.float32),
    )(aggp, d, w0x)


CA = 64
NCA = EPW // CA
CB = 32
NCB = EPW // CB
NSTR = 512
NDW = NPAD // NSTR


def _attn_body(
    src_hbm, dst_hbm, dstf_hbm, etf_hbm, gate_hbm, u_hbm, wd_hbm,
    vt_hbm, a2_hbm,
    wp_hbm, denp_hbm,
    vt_l, a2_l, src_v, dst_v, dstf_c, etf_c, gate_c,
    urows, wdrows, wbuf, denloc,
):
    cid = lax.axis_index("c")
    sid = lax.axis_index("s")
    w = cid * NS + sid
    lanes = lax.iota(jnp.int32, L)
    lane0 = lanes == 0
    zero16 = jnp.zeros((L,), jnp.float32)

    pltpu.sync_copy(vt_hbm, vt_l)
    pltpu.sync_copy(a2_hbm, a2_l)

    def _zd(i, _):
        denloc[pl.ds(i * L, L)] = zero16
        return 0
    lax.fori_loop(0, NPAD // L, _zd, 0)

    def chunk(c, _):
        base = c * CA
        gbase = w * EPW + base
        pltpu.sync_copy(src_hbm.at[pl.ds(gbase, CA)], src_v)
        pltpu.sync_copy(dst_hbm.at[pl.ds(gbase, CA)], dst_v)
        pltpu.sync_copy(dstf_hbm.at[pl.ds(gbase, CA)], dstf_c)
        pltpu.sync_copy(etf_hbm.at[pl.ds(gbase, CA)], etf_c)
        pltpu.sync_copy(gate_hbm.at[pl.ds(gbase, CA)], gate_c)
        pltpu.sync_copy(u_hbm.at[src_v], urows)
        pltpu.sync_copy(wd_hbm.at[dst_v], wdrows)

        def edge(j, _):
            jj = jnp.full((L,), j, jnp.int32)
            et_b = plsc.load_gather(etf_c, [jj]).astype(jnp.int32)
            gate_b = plsc.load_gather(gate_c, [jj])
            dst_b = plsc.load_gather(dstf_c, [jj]).astype(jnp.int32)
            acc = zero16
            for k in range(16):
                c0 = k * L
                z = (urows[j, pl.ds(c0, L)] + wdrows[j, pl.ds(c0, L)]
                     + plsc.load_gather(vt_l, [et_b, c0 + lanes]))
                h = jnp.where(z >= 0, z, 0.2 * z)
                acc = acc + h * a2_l[pl.ds(c0, L)]
            e = jnp.sum(acc)
            pv = jnp.exp(jnp.full((L,), e, jnp.float32))
            wv = pv * gate_b
            plsc.addupdate_scatter(denloc, [dst_b], pv, mask=lane0)
            plsc.store_scatter(wbuf, [base + jj], wv, mask=lane0)
            return 0

        lax.fori_loop(0, CA, edge, 0)
        return 0

    lax.fori_loop(0, NCA, chunk, 0)
    pltpu.sync_copy(wbuf, wp_hbm.at[w, 0, :])
    pltpu.sync_copy(denloc, denp_hbm.at[w, 0, :])


def _denred_body(denp_hbm, dout_hbm, acc, buf):
    cid = lax.axis_index("c")
    sid = lax.axis_index("s")
    w = cid * NS + sid
    base = w * NSTR
    zero16 = jnp.zeros((L,), jnp.float32)

    @pl.when(w < NDW)
    def _():
        def _z(i, _):
            acc[pl.ds(i * L, L)] = zero16
            return 0
        lax.fori_loop(0, NSTR // L, _z, 0)

        def part(p, _):
            pltpu.sync_copy(denp_hbm.at[p, 0, pl.ds(base, NSTR)], buf)
            def add(i, _):
                s = pl.ds(i * L, L)
                acc[s] = acc[s] + buf[s]
                return 0
            lax.fori_loop(0, NSTR // L, add, 0)
            return 0
        lax.fori_loop(0, NW, part, 0)
        pltpu.sync_copy(acc, dout_hbm.at[pl.ds(base, NSTR)])


def _msg_body(
    src_hbm, dst_hbm, etf_hbm, wp_hbm, xv0_hbm, xv1_hbm, rel_hbm, zeros_hbm,
    aggp_hbm,
    rel_l, src_v, dst_v, etf_c, xvh, msg, wbuf,
    agg_sh,
):
    cid = lax.axis_index("c")
    sid = lax.axis_index("s")
    w = cid * NS + sid
    lanes = lax.iota(jnp.int32, L)

    pltpu.sync_copy(rel_hbm, rel_l)
    pltpu.sync_copy(wp_hbm.at[w, 0, :], wbuf)
    xvs = (xv0_hbm, xv1_hbm)

    for h in (0, 1):
        pltpu.sync_copy(zeros_hbm, agg_sh.at[pl.ds(sid * 640, 640), :])
        plsc.subcore_barrier()

        def chunk(c, _, h=h):
            base = c * CB
            gbase = w * EPW + base
            pltpu.sync_copy(src_hbm.at[pl.ds(gbase, CB)], src_v)
            pltpu.sync_copy(dst_hbm.at[pl.ds(gbase, CB)], dst_v)
            pltpu.sync_copy(etf_hbm.at[pl.ds(gbase, CB)], etf_c)
            pltpu.sync_copy(xvs[h].at[src_v], xvh)

            def edge(j, _):
                jj = jnp.full((L,), j, jnp.int32)
                et_b = plsc.load_gather(etf_c, [jj]).astype(jnp.int32)
                wv = plsc.load_gather(wbuf, [base + jj])
                for k in range(8):
                    c0 = k * L
                    msg[j, pl.ds(c0, L)] = (
                        wv * xvh[j, pl.ds(c0, L)]
                        * plsc.load_gather(rel_l,
                                           [et_b, h * 128 + c0 + lanes]))
                return 0

            lax.fori_loop(0, CB, edge, 0)
            pltpu.sync_copy(msg, agg_sh.at[dst_v], add=True)
            return 0

        lax.fori_loop(0, NCB, chunk, 0)
        plsc.subcore_barrier()
        pltpu.sync_copy(agg_sh.at[pl.ds(sid * 640, 640), :],
                        aggp_hbm.at[cid, h, pl.ds(sid * 640, 640), :])
        plsc.subcore_barrier()


_SC_MESH = dict(core_axis_name="c", subcore_axis_name="s",
                num_cores=NC, num_subcores=NS)
_SC_PARAMS = None


def _attn_call(src_p, dst_p, dstf_p, etf_p, gate_p, u, wd, vt, a2):
    f32 = jnp.float32
    i32 = jnp.int32
    k = pl.kernel(
        _attn_body,
        out_type=(
            jax.ShapeDtypeStruct((NW, 1, EPW), f32),
            jax.ShapeDtypeStruct((NW, 1, NPAD), f32),
        ),
        mesh=plsc.VectorSubcoreMesh(**_SC_MESH),
        compiler_params=pltpu.CompilerParams(needs_layout_passes=False),
        scratch_types=[
            pltpu.VMEM((N_REL, 256), f32),
            pltpu.VMEM((256,), f32),
            pltpu.VMEM((CA,), i32),
            pltpu.VMEM((CA,), i32),
            pltpu.VMEM((CA,), f32),
            pltpu.VMEM((CA,), f32),
            pltpu.VMEM((CA,), f32),
            pltpu.VMEM((CA, 256), f32),
            pltpu.VMEM((CA, 256), f32),
            pltpu.VMEM((EPW,), f32),
            pltpu.VMEM((NPAD,), f32),
        ],
    )
    return k(src_p, dst_p, dstf_p, etf_p, gate_p, u, wd, vt, a2)


def _denred_call(denp):
    f32 = jnp.float32
    k = pl.kernel(
        _denred_body,
        out_type=jax.ShapeDtypeStruct((NPAD,), f32),
        mesh=plsc.VectorSubcoreMesh(**_SC_MESH),
        compiler_params=pltpu.CompilerParams(needs_layout_passes=False),
        scratch_types=[
            pltpu.VMEM((NSTR,), f32),
            pltpu.VMEM((NSTR,), f32),
        ],
    )
    return k(denp)


def _msg_call(src_p, dst_p, etf_p, wp, xv0, xv1, rel):
    f32 = jnp.float32
    i32 = jnp.int32
    k = pl.kernel(
        _msg_body,
        out_type=jax.ShapeDtypeStruct((NC, 2, NPAD, H), f32),
        mesh=plsc.VectorSubcoreMesh(**_SC_MESH),
        compiler_params=pltpu.CompilerParams(needs_layout_passes=False),
        scratch_types=[
            pltpu.VMEM((N_REL, 256), f32),
            pltpu.VMEM((CB,), i32),
            pltpu.VMEM((CB,), i32),
            pltpu.VMEM((CB,), f32),
            pltpu.VMEM((CB, H), f32),
            pltpu.VMEM((CB, H), f32),
            pltpu.VMEM((EPW,), f32),
            pltpu.VMEM_SHARED((NPAD, H), f32),
        ],
    )
    zeros = jnp.zeros((640, H), f32)
    return k(src_p, dst_p, etf_p, wp, xv0, xv1, rel, zeros)


def kernel(entity_emb, relation_emb, layer_params, edge_conf, edge_index,
           edge_type):
    f32 = jnp.float32
    i32 = jnp.int32
    src = edge_index[0].astype(i32)
    dst = edge_index[1].astype(i32)
    et = edge_type.astype(i32)
    npad_e = EP - N_EDGES

    src_p = jnp.concatenate([src, jnp.zeros((npad_e,), i32)])
    dst_p = jnp.concatenate([dst, jnp.full((npad_e,), N_NODES, i32)])
    dstf_p = dst_p.astype(f32)
    etf_p = jnp.concatenate([et, jnp.zeros((npad_e,), i32)]).astype(f32)
    conf_p = jnp.concatenate(
        [edge_conf.astype(f32), jnp.zeros((npad_e,), f32)]).reshape(EP // 128,
                                                                    128)
    x = jnp.pad(entity_emb, ((0, NPAD - N_NODES), (0, 0)))
    for p in layer_params:
        a1 = p["a1_w"]
        wcat = jnp.concatenate([a1[0:256], a1[512:768], p["Wv_w"], p["W0_w"]],
                               axis=1)
        bcat = jnp.concatenate(
            [jnp.zeros((512,), f32), p["Wv_b"], p["W0_b"]]).reshape(1, 1024)
        u, wd, xv0, xv1, w0x = _node_matmul(x, wcat, bcat)
        vt = _rel_proj(relation_emb, a1[256:512], p["a1_b"].reshape(1, 256))
        gate = _edge_gate(conf_p, p["cg_w"].reshape(1, 1),
                          p["cg_b"].reshape(1, 1)).reshape(EP)
        a2 = p["a2_w"].reshape(256)
        wp, denp = _attn_call(src_p, dst_p, dstf_p, etf_p, gate, u, wd, vt,
                              a2)
        den = _denred_call(denp)
        aggp = _msg_call(src_p, dst_p, etf_p, wp, xv0, xv1, relation_emb)
        x = _finalize(aggp, den.reshape(NPAD, 1), w0x)
    return x[:N_NODES]


# speedup vs baseline: 1.6990x; 1.1850x over previous
import functools

import jax
import jax.numpy as jnp
from jax import lax
from jax.experimental import pallas as pl
from jax.experimental.pallas import tpu as pltpu
from jax.experimental.pallas import tpu_sc as plsc

N_NODES = 10000
N_EDGES = 160000
D = 256
H = 128
N_REL = 64

NC = 2
NS = 16
NW = NC * NS
L = 16

C = 32
NCHUNK = 160
Q = 64
EPW = C * NCHUNK
EP = EPW * NW
NPAD = 10240
MBLK = 1024


def _mm_body(x_ref, w_ref, b_ref, u_ref, wd_ref, xv0_ref, xv1_ref, w0x_ref):
    p = jnp.dot(x_ref[...], w_ref[...], preferred_element_type=jnp.float32)
    p = p + b_ref[...]
    u_ref[...] = p[:, 0:256]
    wd_ref[...] = p[:, 256:512]
    xv0_ref[...] = p[:, 512:640]
    xv1_ref[...] = p[:, 640:768]
    w0x_ref[...] = p[:, 768:1024]


def _node_matmul(x, wcat, bcat):
    n = NPAD // MBLK
    outs = (
        jax.ShapeDtypeStruct((NPAD, 256), jnp.float32),
        jax.ShapeDtypeStruct((NPAD, 256), jnp.float32),
        jax.ShapeDtypeStruct((NPAD, H), jnp.float32),
        jax.ShapeDtypeStruct((NPAD, H), jnp.float32),
        jax.ShapeDtypeStruct((NPAD, 256), jnp.float32),
    )
    return pl.pallas_call(
        _mm_body,
        grid=(n,),
        in_specs=[
            pl.BlockSpec((MBLK, 256), lambda i: (i, 0)),
            pl.BlockSpec((256, 1024), lambda i: (0, 0)),
            pl.BlockSpec((1, 1024), lambda i: (0, 0)),
        ],
        out_specs=[
            pl.BlockSpec((MBLK, 256), lambda i: (i, 0)),
            pl.BlockSpec((MBLK, 256), lambda i: (i, 0)),
            pl.BlockSpec((MBLK, H), lambda i: (i, 0)),
            pl.BlockSpec((MBLK, H), lambda i: (i, 0)),
            pl.BlockSpec((MBLK, 256), lambda i: (i, 0)),
        ],
        out_shape=outs,
    )(x, wcat, bcat)


def _vt_body(r_ref, a_ref, b_ref, o_ref):
    o_ref[...] = (
        jnp.dot(r_ref[...], a_ref[...], preferred_element_type=jnp.float32)
        + b_ref[...]
    )


def _rel_proj(rel_emb, a_r, b1):
    return pl.pallas_call(
        _vt_body,
        out_shape=jax.ShapeDtypeStruct((N_REL, 256), jnp.float32),
    )(rel_emb, a_r, b1)


def _gate_body(c_ref, a_ref, b_ref, o_ref):
    t = c_ref[...] * a_ref[0, 0] + b_ref[0, 0]
    o_ref[...] = 1.0 / (1.0 + jnp.exp(-t))


def _edge_gate(conf_p, cg_w, cg_b):
    return pl.pallas_call(
        _gate_body,
        out_shape=jax.ShapeDtypeStruct(conf_p.shape, jnp.float32),
    )(conf_p, cg_w, cg_b)


def _fin_body(agg_ref, d_ref, w0x_ref, o_ref):
    den = d_ref[...] + 1e-16
    s = agg_ref[0, 0] + agg_ref[1, 0]
    o_ref[...] = jnp.maximum(s / den + w0x_ref[...], 0.0)


def _finalize(aggp, d, w0x):
    n = NPAD // MBLK
    return pl.pallas_call(
        _fin_body,
        grid=(n, 2),
        in_specs=[
            pl.BlockSpec((2, 1, MBLK, H), lambda i, h: (0, h, i, 0)),
            pl.BlockSpec((MBLK, 1), lambda i, h: (i, 0)),
            pl.BlockSpec((MBLK, H), lambda i, h: (i, h)),
        ],
        out_specs=pl.BlockSpec((MBLK, H), lambda i, h: (i, h)),
        out_shape=jax.ShapeDtypeStruct((NPAD, 256), jnp.float32),
    )(aggp, d, w0x)


CA = 128
NCA = EPW // CA
CB = 64
NCB = EPW // CB
NSTR = 512
NDW = NPAD // NSTR


def _attn_body(
    src_hbm, dst_hbm, dstf_hbm, etf_hbm, gate_hbm, u_hbm, wd_hbm,
    vt_hbm, a2_hbm,
    wp_hbm, denp_hbm,
    vt_l, a2_l, src_v, dst_v, dstf_c, etf_c, gate_c,
    urows, wdrows, wbuf, denloc,
):
    cid = lax.axis_index("c")
    sid = lax.axis_index("s")
    w = cid * NS + sid
    lanes = lax.iota(jnp.int32, L)
    lane0 = lanes == 0
    zero16 = jnp.zeros((L,), jnp.float32)

    pltpu.sync_copy(vt_hbm, vt_l)
    pltpu.sync_copy(a2_hbm, a2_l)

    def _zd(i, _):
        denloc[pl.ds(i * L, L)] = zero16
        return 0
    lax.fori_loop(0, NPAD // L, _zd, 0)

    def chunk(c, _):
        base = c * CA
        gbase = w * EPW + base
        pltpu.sync_copy(src_hbm.at[pl.ds(gbase, CA)], src_v)
        pltpu.sync_copy(dst_hbm.at[pl.ds(gbase, CA)], dst_v)
        pltpu.sync_copy(dstf_hbm.at[pl.ds(gbase, CA)], dstf_c)
        pltpu.sync_copy(etf_hbm.at[pl.ds(gbase, CA)], etf_c)
        pltpu.sync_copy(gate_hbm.at[pl.ds(gbase, CA)], gate_c)
        pltpu.sync_copy(u_hbm.at[src_v], urows)
        pltpu.sync_copy(wd_hbm.at[dst_v], wdrows)

        def edge(j, _):
            jj = jnp.full((L,), j, jnp.int32)
            et_b = plsc.load_gather(etf_c, [jj]).astype(jnp.int32)
            gate_b = plsc.load_gather(gate_c, [jj])
            dst_b = plsc.load_gather(dstf_c, [jj]).astype(jnp.int32)
            acc = zero16
            for k in range(16):
                c0 = k * L
                z = (urows[j, pl.ds(c0, L)] + wdrows[j, pl.ds(c0, L)]
                     + plsc.load_gather(vt_l, [et_b, c0 + lanes]))
                h = jnp.where(z >= 0, z, 0.2 * z)
                acc = acc + h * a2_l[pl.ds(c0, L)]
            e = jnp.sum(acc)
            pv = jnp.exp(jnp.full((L,), e, jnp.float32))
            wv = pv * gate_b
            plsc.addupdate_scatter(denloc, [dst_b], pv, mask=lane0)
            plsc.store_scatter(wbuf, [base + jj], wv, mask=lane0)
            return 0

        lax.fori_loop(0, CA, edge, 0)
        return 0

    lax.fori_loop(0, NCA, chunk, 0)
    pltpu.sync_copy(wbuf, wp_hbm.at[w, 0, :])
    pltpu.sync_copy(denloc, denp_hbm.at[w, 0, :])


def _denred_body(denp_hbm, dout_hbm, acc, buf):
    cid = lax.axis_index("c")
    sid = lax.axis_index("s")
    w = cid * NS + sid
    base = w * NSTR
    zero16 = jnp.zeros((L,), jnp.float32)

    @pl.when(w < NDW)
    def _():
        def _z(i, _):
            acc[pl.ds(i * L, L)] = zero16
            return 0
        lax.fori_loop(0, NSTR // L, _z, 0)

        def part(p, _):
            pltpu.sync_copy(denp_hbm.at[p, 0, pl.ds(base, NSTR)], buf)
            def add(i, _):
                s = pl.ds(i * L, L)
                acc[s] = acc[s] + buf[s]
                return 0
            lax.fori_loop(0, NSTR // L, add, 0)
            return 0
        lax.fori_loop(0, NW, part, 0)
        pltpu.sync_copy(acc, dout_hbm.at[pl.ds(base, NSTR)])


def _msg_body(
    src_hbm, dst_hbm, etf_hbm, wp_hbm, xv0_hbm, xv1_hbm, rel_hbm, zeros_hbm,
    aggp_hbm,
    rel_l, src_v, dst_v, etf_all, xvh, msg, wbuf,
    agg_sh,
):
    cid = lax.axis_index("c")
    sid = lax.axis_index("s")
    w = cid * NS + sid
    lanes = lax.iota(jnp.int32, L)

    pltpu.sync_copy(rel_hbm, rel_l)
    pltpu.sync_copy(wp_hbm.at[w, 0, :], wbuf)
    pltpu.sync_copy(etf_hbm.at[pl.ds(w * EPW, EPW)], etf_all)
    xvs = (xv0_hbm, xv1_hbm)

    for h in (0, 1):
        pltpu.sync_copy(zeros_hbm, agg_sh.at[pl.ds(sid * 640, 640), :])
        plsc.subcore_barrier()

        def chunk(c, _, h=h):
            base = c * CB
            gbase = w * EPW + base
            pltpu.sync_copy(src_hbm.at[pl.ds(gbase, CB)], src_v)
            pltpu.sync_copy(dst_hbm.at[pl.ds(gbase, CB)], dst_v)
            pltpu.sync_copy(xvs[h].at[src_v], xvh)

            def edge(j, _):
                jj = jnp.full((L,), j, jnp.int32)
                et_b = plsc.load_gather(etf_all, [base + jj]).astype(jnp.int32)
                wv = plsc.load_gather(wbuf, [base + jj])
                for k in range(8):
                    c0 = k * L
                    msg[j, pl.ds(c0, L)] = (
                        wv * xvh[j, pl.ds(c0, L)]
                        * plsc.load_gather(rel_l,
                                           [et_b, h * 128 + c0 + lanes]))
                return 0

            lax.fori_loop(0, CB, edge, 0)
            pltpu.sync_copy(msg, agg_sh.at[dst_v], add=True)
            return 0

        lax.fori_loop(0, NCB, chunk, 0)
        plsc.subcore_barrier()
        pltpu.sync_copy(agg_sh.at[pl.ds(sid * 640, 640), :],
                        aggp_hbm.at[cid, h, pl.ds(sid * 640, 640), :])
        plsc.subcore_barrier()


_SC_MESH = dict(core_axis_name="c", subcore_axis_name="s",
                num_cores=NC, num_subcores=NS)
_SC_PARAMS = None


def _attn_call(src_p, dst_p, dstf_p, etf_p, gate_p, u, wd, vt, a2):
    f32 = jnp.float32
    i32 = jnp.int32
    k = pl.kernel(
        _attn_body,
        out_type=(
            jax.ShapeDtypeStruct((NW, 1, EPW), f32),
            jax.ShapeDtypeStruct((NW, 1, NPAD), f32),
        ),
        mesh=plsc.VectorSubcoreMesh(**_SC_MESH),
        compiler_params=pltpu.CompilerParams(needs_layout_passes=False),
        scratch_types=[
            pltpu.VMEM((N_REL, 256), f32),
            pltpu.VMEM((256,), f32),
            pltpu.VMEM((CA,), i32),
            pltpu.VMEM((CA,), i32),
            pltpu.VMEM((CA,), f32),
            pltpu.VMEM((CA,), f32),
            pltpu.VMEM((CA,), f32),
            pltpu.VMEM((CA, 256), f32),
            pltpu.VMEM((CA, 256), f32),
            pltpu.VMEM((EPW,), f32),
            pltpu.VMEM((NPAD,), f32),
        ],
    )
    return k(src_p, dst_p, dstf_p, etf_p, gate_p, u, wd, vt, a2)


def _denred_call(denp):
    f32 = jnp.float32
    k = pl.kernel(
        _denred_body,
        out_type=jax.ShapeDtypeStruct((NPAD,), f32),
        mesh=plsc.VectorSubcoreMesh(**_SC_MESH),
        compiler_params=pltpu.CompilerParams(needs_layout_passes=False),
        scratch_types=[
            pltpu.VMEM((NSTR,), f32),
            pltpu.VMEM((NSTR,), f32),
        ],
    )
    return k(denp)


def _msg_call(src_p, dst_p, etf_p, wp, xv0, xv1, rel):
    f32 = jnp.float32
    i32 = jnp.int32
    k = pl.kernel(
        _msg_body,
        out_type=jax.ShapeDtypeStruct((NC, 2, NPAD, H), f32),
        mesh=plsc.VectorSubcoreMesh(**_SC_MESH),
        compiler_params=pltpu.CompilerParams(needs_layout_passes=False),
        scratch_types=[
            pltpu.VMEM((N_REL, 256), f32),
            pltpu.VMEM((CB,), i32),
            pltpu.VMEM((CB,), i32),
            pltpu.VMEM((EPW,), f32),
            pltpu.VMEM((CB, H), f32),
            pltpu.VMEM((CB, H), f32),
            pltpu.VMEM((EPW,), f32),
            pltpu.VMEM_SHARED((NPAD, H), f32),
        ],
    )
    zeros = jnp.zeros((640, H), f32)
    return k(src_p, dst_p, etf_p, wp, xv0, xv1, rel, zeros)


def kernel(entity_emb, relation_emb, layer_params, edge_conf, edge_index,
           edge_type):
    f32 = jnp.float32
    i32 = jnp.int32
    src = edge_index[0].astype(i32)
    dst = edge_index[1].astype(i32)
    et = edge_type.astype(i32)
    npad_e = EP - N_EDGES

    src_p = jnp.concatenate([src, jnp.zeros((npad_e,), i32)])
    dst_p = jnp.concatenate([dst, jnp.full((npad_e,), N_NODES, i32)])
    dstf_p = dst_p.astype(f32)
    etf_p = jnp.concatenate([et, jnp.zeros((npad_e,), i32)]).astype(f32)
    conf_p = jnp.concatenate(
        [edge_conf.astype(f32), jnp.zeros((npad_e,), f32)]).reshape(EP // 128,
                                                                    128)
    x = jnp.pad(entity_emb, ((0, NPAD - N_NODES), (0, 0)))
    for p in layer_params:
        a1 = p["a1_w"]
        wcat = jnp.concatenate([a1[0:256], a1[512:768], p["Wv_w"], p["W0_w"]],
                               axis=1)
        bcat = jnp.concatenate(
            [jnp.zeros((512,), f32), p["Wv_b"], p["W0_b"]]).reshape(1, 1024)
        u, wd, xv0, xv1, w0x = _node_matmul(x, wcat, bcat)
        vt = _rel_proj(relation_emb, a1[256:512], p["a1_b"].reshape(1, 256))
        gate = _edge_gate(conf_p, p["cg_w"].reshape(1, 1),
                          p["cg_b"].reshape(1, 1)).reshape(EP)
        a2 = p["a2_w"].reshape(256)
        wp, denp = _attn_call(src_p, dst_p, dstf_p, etf_p, gate, u, wd, vt,
                              a2)
        den = _denred_call(denp)
        aggp = _msg_call(src_p, dst_p, etf_p, wp, xv0, xv1, relation_emb)
        x = _finalize(aggp, den.reshape(NPAD, 1), w0x)
    return x[:N_NODES]
